# baseline (device time: 15438 ns/iter reference)
import os

import jax
import jax.numpy as jnp
from jax import lax
from jax.experimental import pallas as pl
from jax.experimental.pallas import tpu as pltpu

N_DEV = 32
WIN = 128
_NO_RDMA = os.environ.get("SCBAND_NO_RDMA") == "1"


def kernel(x, Wq, K_ext, V_ext, Wo):
    B, S, Hq, Dh = K_ext.shape
    Dm = x.shape[-1]
    Dq = Hq * Dh
    S2 = S + 2 * WIN
    bf = jnp.bfloat16

    K2 = K_ext.reshape(B, S, Dq)
    V2 = V_ext.reshape(B, S, Dq)

    def body(x_ref, wq_ref, k_ref, v_ref, wo_ref, out_ref,
             kbuf, vbuf, ctxbuf, send_sems, recv_sems):
        my = lax.axis_index("i")
        left = my - 1
        right = my + 1

        zero = jnp.zeros((B, WIN, Dq), bf)
        kbuf[:, :WIN] = zero
        vbuf[:, :WIN] = zero
        kbuf[:, WIN + S:] = zero
        vbuf[:, WIN + S:] = zero

        if not _NO_RDMA:
            barrier = pltpu.get_barrier_semaphore()

            @pl.when(my > 0)
            def _():
                pl.semaphore_signal(barrier, inc=1, device_id=(left,),
                                    device_id_type=pl.DeviceIdType.MESH)

            @pl.when(my == 0)
            def _():
                pl.semaphore_signal(barrier, inc=1)

            @pl.when(my < N_DEV - 1)
            def _():
                pl.semaphore_signal(barrier, inc=1, device_id=(right,),
                                    device_id_type=pl.DeviceIdType.MESH)

            @pl.when(my == N_DEV - 1)
            def _():
                pl.semaphore_signal(barrier, inc=1)

        kbuf[:, WIN:WIN + S] = k_ref[...].astype(bf)

        if not _NO_RDMA:
            pl.semaphore_wait(barrier, 2)

        def halo(buf, src_lo, dst_lo, s_slot, r_slot, dev):
            return pltpu.make_async_remote_copy(
                src_ref=buf.at[:, src_lo:src_lo + WIN],
                dst_ref=buf.at[:, dst_lo:dst_lo + WIN],
                send_sem=send_sems.at[s_slot],
                recv_sem=recv_sems.at[r_slot],
                device_id=(dev,),
                device_id_type=pl.DeviceIdType.MESH,
            )

        if not _NO_RDMA:
            @pl.when(my > 0)
            def _():
                halo(kbuf, WIN, WIN + S, 0, 2, left).start()

            @pl.when(my < N_DEV - 1)
            def _():
                halo(kbuf, S, 0, 2, 0, right).start()

        vbuf[:, WIN:WIN + S] = v_ref[...].astype(bf)

        if not _NO_RDMA:
            @pl.when(my > 0)
            def _():
                halo(vbuf, WIN, WIN + S, 1, 3, left).start()

            @pl.when(my < N_DEV - 1)
            def _():
                halo(vbuf, S, 0, 3, 1, right).start()

        wq = wq_ref[...].astype(bf)
        qs = [(jnp.dot(x_ref[b].astype(bf), wq,
                       preferred_element_type=jnp.float32)
               * 0.125).astype(bf)
              for b in range(B)]

        if not _NO_RDMA:
            @pl.when(my > 0)
            def _():
                halo(kbuf, WIN, 0, 0, 0, left).wait_recv()
                halo(vbuf, WIN, 0, 1, 1, left).wait_recv()

            @pl.when(my < N_DEV - 1)
            def _():
                halo(kbuf, WIN, WIN + S, 2, 2, right).wait_recv()
                halo(vbuf, WIN, WIN + S, 3, 3, right).wait_recv()

        qi = lax.broadcasted_iota(jnp.int32, (S, S2), 0)
        kj = lax.broadcasted_iota(jnp.int32, (S, S2), 1)
        in_win = jnp.abs(kj - WIN - qi) <= WIN
        kglob = kj + my * S - WIN
        mask = in_win & (kglob >= 0) & (kglob < N_DEV * S)
        bias = jnp.where(mask, jnp.float32(0.0), jnp.float32(-1e9))

        for b in range(B):
            for h in range(Hq):
                q = qs[b][:, h * Dh:(h + 1) * Dh]
                k = kbuf[b, :, h * Dh:(h + 1) * Dh]
                s = lax.dot_general(
                    q, k, (((1,), (1,)), ((), ())),
                    preferred_element_type=jnp.float32)
                w = jnp.exp(s + bias)
                denom = jnp.sum(w, axis=-1, keepdims=True)
                v = vbuf[b, :, h * Dh:(h + 1) * Dh]
                ctx = jnp.dot(w.astype(bf), v,
                              preferred_element_type=jnp.float32)
                ctx = ctx * (1.0 / denom)
                ctxbuf[b, :, h * Dh:(h + 1) * Dh] = ctx.astype(bf)

        wo = wo_ref[...].astype(bf)
        for b in range(B):
            out_ref[b] = jnp.dot(ctxbuf[b], wo,
                                 preferred_element_type=jnp.float32)

        if not _NO_RDMA:
            @pl.when(my > 0)
            def _():
                halo(kbuf, WIN, WIN + S, 0, 2, left).wait_send()
                halo(vbuf, WIN, WIN + S, 1, 3, left).wait_send()

            @pl.when(my < N_DEV - 1)
            def _():
                halo(kbuf, S, 0, 2, 0, right).wait_send()
                halo(vbuf, S, 0, 3, 1, right).wait_send()

    return pl.pallas_call(
        body,
        out_shape=jax.ShapeDtypeStruct((B, S, Dm), jnp.float32),
        in_specs=[pl.BlockSpec(memory_space=pltpu.VMEM)] * 5,
        out_specs=pl.BlockSpec(memory_space=pltpu.VMEM),
        scratch_shapes=[
            pltpu.VMEM((B, S2, Dq), bf),
            pltpu.VMEM((B, S2, Dq), bf),
            pltpu.VMEM((B, S, Dq), bf),
            pltpu.SemaphoreType.DMA((4,)),
            pltpu.SemaphoreType.DMA((4,)),
        ],
        compiler_params=pltpu.CompilerParams(
            collective_id=None if _NO_RDMA else 0),
    )(x, Wq, K2, V2, Wo)


# device time: 7901 ns/iter; 1.9539x vs baseline; 1.9539x over previous
import os

import jax
import jax.numpy as jnp
from jax import lax
from jax.experimental import pallas as pl
from jax.experimental.pallas import tpu as pltpu

N_DEV = 32
WIN = 128
_NO_RDMA = os.environ.get("SCBAND_NO_RDMA") == "1"


def kernel(x, Wq, K_ext, V_ext, Wo):
    B, S, Hq, Dh = K_ext.shape
    Dm = x.shape[-1]
    Dq = Hq * Dh
    S2 = S + 2 * WIN
    bf = jnp.bfloat16

    K2 = K_ext.reshape(B, S, Dq)
    V2 = V_ext.reshape(B, S, Dq)

    def body(x_ref, wq_ref, k_ref, v_ref, wo_ref, out_ref,
             kbuf, vbuf, ctxbuf, send_sems, recv_sems):
        my = lax.axis_index("i")
        left = my - 1
        right = my + 1

        zero = jnp.zeros((B, WIN, Dq), bf)
        kbuf[:, :WIN] = zero
        vbuf[:, :WIN] = zero
        kbuf[:, WIN + S:] = zero
        vbuf[:, WIN + S:] = zero

        if not _NO_RDMA:
            barrier = pltpu.get_barrier_semaphore()

            @pl.when(my > 0)
            def _():
                pl.semaphore_signal(barrier, inc=1, device_id=(left,),
                                    device_id_type=pl.DeviceIdType.MESH)

            @pl.when(my == 0)
            def _():
                pl.semaphore_signal(barrier, inc=1)

            @pl.when(my < N_DEV - 1)
            def _():
                pl.semaphore_signal(barrier, inc=1, device_id=(right,),
                                    device_id_type=pl.DeviceIdType.MESH)

            @pl.when(my == N_DEV - 1)
            def _():
                pl.semaphore_signal(barrier, inc=1)

        kbuf[:, WIN:WIN + S] = k_ref[...].astype(bf)

        if not _NO_RDMA:
            pl.semaphore_wait(barrier, 2)

        def halo(buf, src_lo, dst_lo, s_slot, r_slot, dev):
            return pltpu.make_async_remote_copy(
                src_ref=buf.at[:, src_lo:src_lo + WIN],
                dst_ref=buf.at[:, dst_lo:dst_lo + WIN],
                send_sem=send_sems.at[s_slot],
                recv_sem=recv_sems.at[r_slot],
                device_id=(dev,),
                device_id_type=pl.DeviceIdType.MESH,
            )

        if not _NO_RDMA:
            @pl.when(my > 0)
            def _():
                halo(kbuf, WIN, WIN + S, 0, 2, left).start()

            @pl.when(my < N_DEV - 1)
            def _():
                halo(kbuf, S, 0, 2, 0, right).start()

        vbuf[:, WIN:WIN + S] = v_ref[...].astype(bf)

        if not _NO_RDMA:
            @pl.when(my > 0)
            def _():
                halo(vbuf, WIN, WIN + S, 1, 3, left).start()

            @pl.when(my < N_DEV - 1)
            def _():
                halo(vbuf, S, 0, 3, 1, right).start()

        wq = wq_ref[...].astype(bf)
        qs = [(jnp.dot(x_ref[b].astype(bf), wq,
                       preferred_element_type=jnp.float32)
               * 0.125).astype(bf)
              for b in range(B)]

        zero32 = jnp.float32(0.0)
        neg = jnp.float32(-1e9)

        qi_c = lax.broadcasted_iota(jnp.int32, (S, S), 0)
        kj_c = lax.broadcasted_iota(jnp.int32, (S, S), 1)
        bias_c = jnp.where(jnp.abs(kj_c - qi_c) <= WIN, zero32, neg)

        qi_h = lax.broadcasted_iota(jnp.int32, (WIN, WIN), 0)
        kj_h = lax.broadcasted_iota(jnp.int32, (WIN, WIN), 1)
        bias_l = (jnp.where(kj_h >= qi_h, zero32, neg)
                  + jnp.where(my > 0, zero32, neg))
        bias_r = (jnp.where(kj_h <= qi_h, zero32, neg)
                  + jnp.where(my < N_DEV - 1, zero32, neg))

        def attn_block(q, k, v, bias):
            s = lax.dot_general(q, k, (((1,), (1,)), ((), ())),
                                preferred_element_type=jnp.float32)
            w = jnp.exp(s + bias)
            ctx = jnp.dot(w.astype(bf), v,
                          preferred_element_type=jnp.float32)
            return ctx, jnp.sum(w, axis=-1, keepdims=True)

        center = []
        for b in range(B):
            for h in range(Hq):
                sl = slice(h * Dh, (h + 1) * Dh)
                center.append(attn_block(
                    qs[b][:, sl], kbuf[b, WIN:WIN + S, sl],
                    vbuf[b, WIN:WIN + S, sl], bias_c))

        if not _NO_RDMA:
            @pl.when(my > 0)
            def _():
                halo(kbuf, WIN, 0, 0, 0, left).wait_recv()
                halo(vbuf, WIN, 0, 1, 1, left).wait_recv()

            @pl.when(my < N_DEV - 1)
            def _():
                halo(kbuf, WIN, WIN + S, 2, 2, right).wait_recv()
                halo(vbuf, WIN, WIN + S, 3, 3, right).wait_recv()

        for b in range(B):
            for h in range(Hq):
                sl = slice(h * Dh, (h + 1) * Dh)
                ctx_c, den_c = center[b * Hq + h]
                ctx_l, den_l = attn_block(
                    qs[b][:WIN, sl], kbuf[b, :WIN, sl],
                    vbuf[b, :WIN, sl], bias_l)
                ctx_r, den_r = attn_block(
                    qs[b][WIN:, sl], kbuf[b, WIN + S:, sl],
                    vbuf[b, WIN + S:, sl], bias_r)
                top = (ctx_c[:WIN] + ctx_l) * (1.0 / (den_c[:WIN] + den_l))
                bot = (ctx_c[WIN:] + ctx_r) * (1.0 / (den_c[WIN:] + den_r))
                ctxbuf[b, :WIN, sl] = top.astype(bf)
                ctxbuf[b, WIN:, sl] = bot.astype(bf)

        wo = wo_ref[...].astype(bf)
        for b in range(B):
            out_ref[b] = jnp.dot(ctxbuf[b], wo,
                                 preferred_element_type=jnp.float32)

        if not _NO_RDMA:
            @pl.when(my > 0)
            def _():
                halo(kbuf, WIN, WIN + S, 0, 2, left).wait_send()
                halo(vbuf, WIN, WIN + S, 1, 3, left).wait_send()

            @pl.when(my < N_DEV - 1)
            def _():
                halo(kbuf, S, 0, 2, 0, right).wait_send()
                halo(vbuf, S, 0, 3, 1, right).wait_send()

    return pl.pallas_call(
        body,
        out_shape=jax.ShapeDtypeStruct((B, S, Dm), jnp.float32),
        in_specs=[pl.BlockSpec(memory_space=pltpu.VMEM)] * 5,
        out_specs=pl.BlockSpec(memory_space=pltpu.VMEM),
        scratch_shapes=[
            pltpu.VMEM((B, S2, Dq), bf),
            pltpu.VMEM((B, S2, Dq), bf),
            pltpu.VMEM((B, S, Dq), bf),
            pltpu.SemaphoreType.DMA((4,)),
            pltpu.SemaphoreType.DMA((4,)),
        ],
        compiler_params=pltpu.CompilerParams(
            collective_id=None if _NO_RDMA else 0),
    )(x, Wq, K2, V2, Wo)
